# baseline (speedup 1.0000x reference)
import functools

import jax
import jax.numpy as jnp
from jax import lax
from jax.experimental import pallas as pl
from jax.experimental.pallas import tpu as pltpu
from jax.experimental.pallas import tpu_sc as plsc

NC = 2
NS = 16
LANES = 16

K_EDGES = 128
D = 128
DH = D // NC


def _support_tc(feats, W_enc_stacked, b_enc_stacked, W_gcn, b1, b2, blk,
                n_pad):
  n_total = feats.shape[0]
  n_blocks = n_total // blk

  def tmap(i):
    return jnp.where(i >= b2, 2, jnp.where(i >= b1, 1, 0))

  def body(x_ref, w_ref, b_ref, wg_ref, out_ref):
    x = x_ref[...]
    enc = jnp.maximum(
        jnp.dot(x, w_ref[0], preferred_element_type=jnp.float32) + b_ref[0],
        0.0)
    s = jnp.dot(enc, wg_ref[...], preferred_element_type=jnp.float32)
    out_ref[0] = s[:, :DH]
    out_ref[1] = s[:, DH:]

  return pl.pallas_call(
      body,
      grid=(n_blocks,),
      in_specs=[
          pl.BlockSpec((blk, D), lambda i: (i, 0)),
          pl.BlockSpec((1, D, D), lambda i: (tmap(i), 0, 0)),
          pl.BlockSpec((1, 1, D), lambda i: (tmap(i), 0, 0)),
          pl.BlockSpec((D, D), lambda i: (0, 0)),
      ],
      out_specs=pl.BlockSpec((NC, blk, DH), lambda i: (0, i, 0)),
      out_shape=jax.ShapeDtypeStruct((NC, n_pad, DH), jnp.float32),
  )(feats, W_enc_stacked, b_enc_stacked, W_gcn)


def _edge_aggregate_sc(support2, src2d, dst2d, wgt2d, n_acc, ch_per_tile):
  rows_per_sub = n_acc // NS
  p_ch = 16
  n_phase = ch_per_tile // p_ch
  hb = DH // LANES

  mesh = plsc.VectorSubcoreMesh(core_axis_name="c", subcore_axis_name="s")

  @functools.partial(
      pl.kernel,
      out_type=jax.ShapeDtypeStruct((NC, n_acc, DH), jnp.float32),
      mesh=mesh,
      compiler_params=pltpu.CompilerParams(use_tc_tiling_on_sc=False),
      scratch_types=[
          pltpu.VMEM((p_ch, K_EDGES), jnp.int32),
          pltpu.VMEM((p_ch, K_EDGES), jnp.int32),
          pltpu.VMEM((p_ch, K_EDGES), jnp.int32),
          pltpu.VMEM((p_ch, K_EDGES), jnp.int32),
          pltpu.VMEM((p_ch, K_EDGES), jnp.float32),
          pltpu.VMEM((p_ch, K_EDGES), jnp.float32),
          pltpu.VMEM((K_EDGES, DH), jnp.float32),
          pltpu.VMEM((K_EDGES, DH), jnp.float32),
          pltpu.VMEM((K_EDGES, DH), jnp.float32),
          pltpu.VMEM((K_EDGES, DH), jnp.float32),
          pltpu.VMEM_SHARED((n_acc, DH), jnp.float32),
          pltpu.VMEM_SHARED((n_acc, DH), jnp.float32),
          pltpu.SemaphoreType.DMA,
          pltpu.SemaphoreType.DMA,
          pltpu.SemaphoreType.DMA,
          pltpu.SemaphoreType.DMA,
          pltpu.SemaphoreType.DMA,
          pltpu.SemaphoreType.DMA,
      ],
  )
  def sc_kernel(sup_hbm, sup_flat, src_hbm, dst_hbm, wgt_hbm, out_hbm,
                srcb0, srcb1, dstb0, dstb1, wgtb0, wgtb1,
                gbuf0, gbuf1, sbuf0, sbuf1,
                sup_sp, acc, gsem0, gsem1, ssem0, ssem1, slsem0, slsem1):
    cid = lax.axis_index("c")
    sid = lax.axis_index("s")
    gbufs, sbufs = (gbuf0, gbuf1), (sbuf0, sbuf1)
    gsems, ssems = (gsem0, gsem1), (ssem0, ssem1)
    srcbs, dstbs, wgtbs = (srcb0, srcb1), (dstb0, dstb1), (wgtb0, wgtb1)
    slsems = (slsem0, slsem1)
    gdummy = sup_hbm.at[0, pl.ds(0, K_EDGES)]
    sdummy_i = src_hbm.at[pl.ds(0, p_ch)]
    sdummy_f = wgt_hbm.at[pl.ds(0, p_ch)]
    base = sid * rows_per_sub

    pltpu.sync_copy(sup_hbm.at[cid, pl.ds(base, rows_per_sub)],
                    sup_sp.at[pl.ds(base, rows_per_sub)])

    def zfill(i, carry):
      for r in range(hb):
        gbuf0[i, pl.ds(r * LANES, LANES)] = jnp.zeros((LANES,), jnp.float32)
      return carry

    lax.fori_loop(0, K_EDGES, zfill, 0)
    nfull = rows_per_sub // K_EDGES
    rem = rows_per_sub - nfull * K_EDGES
    for kb in range(nfull):
      pltpu.sync_copy(gbuf0.at[pl.ds(0, K_EDGES)],
                      acc.at[pl.ds(base + kb * K_EDGES, K_EDGES)])
    if rem:
      pltpu.sync_copy(gbuf0.at[pl.ds(0, rem)],
                      acc.at[pl.ds(base + nfull * K_EDGES, rem)])
    plsc.subcore_barrier()

    def stage(pi, q):
      eb = sid * ch_per_tile + pi * p_ch
      pltpu.async_copy(src_hbm.at[pl.ds(eb, p_ch)], srcbs[q], slsems[q])
      pltpu.async_copy(dst_hbm.at[pl.ds(eb, p_ch)], dstbs[q], slsems[q])
      pltpu.async_copy(wgt_hbm.at[pl.ds(eb, p_ch)], wgtbs[q], slsems[q])

    def stage_wait(q):
      pltpu.make_async_copy(sdummy_i, srcbs[q], slsems[q]).wait()
      pltpu.make_async_copy(sdummy_i, dstbs[q], slsems[q]).wait()
      pltpu.make_async_copy(sdummy_f, wgtbs[q], slsems[q]).wait()
      off = cid * n_acc
      sb = srcbs[q]

      def arow(r2, carry):
        for g in range(K_EDGES // LANES):
          sl = pl.ds(g * LANES, LANES)
          sb[2 * r2, sl] = sb[2 * r2, sl] + off
        return carry

      lax.fori_loop(0, p_ch // 2, arow, 0)

    def issue_gather(src_row_ref, b):
      if b == 0:
        pltpu.async_copy(sup_flat.at[src_row_ref], gbufs[0], gsems[0])
      else:
        pltpu.async_copy(sup_sp.at[src_row_ref], gbufs[1], gsems[1])

    def scale_chunk(wgt_v, gbuf, sbuf, j):
      def gbody(g, c2):
        w16 = wgt_v[j, pl.ds(g * LANES, LANES)]
        for i in range(LANES):
          e = g * LANES + i
          w = w16[i]
          for r in range(hb):
            sl = pl.ds(r * LANES, LANES)
            sbuf[e, sl] = gbuf[e, sl] * w
        return c2

      lax.fori_loop(0, K_EDGES // LANES, gbody, 0)

    stage(0, 0)
    stage_wait(0)
    stage(1, 1)
    issue_gather(srcb0.at[0], 0)
    issue_gather(srcb0.at[1], 1)

    n_pp = n_phase // 2

    def double_phase(pp, carry):
      for sub in range(2):
        q = sub
        p_dyn = 2 * pp + sub
        src_v, dst_v, wgt_v = srcbs[q], dstbs[q], wgtbs[q]
        last_pp = pp >= n_pp - 1

        for b in range(2):
          pltpu.make_async_copy(gdummy, gbufs[b], gsems[b]).wait()
          if sub == 0:
            @pl.when(pp > 0)
            def _():
              pltpu.make_async_copy(gdummy, sbufs[b], ssems[b]).wait()
          else:
            pltpu.make_async_copy(gdummy, sbufs[b], ssems[b]).wait()
          scale_chunk(wgt_v, gbufs[b], sbufs[b], b)
          pltpu.async_copy(sbufs[b], acc.at[dst_v.at[b]], ssems[b], add=True)
          issue_gather(src_v.at[b + 2], b)
        if sub == 0:
          @pl.when(pp > 0)
          def _():
            stage(p_dyn + 1, 1)
        else:
          @pl.when(jnp.logical_not(last_pp))
          def _():
            stage(p_dyn + 1, 0)

        def chunk_pair(jj, carry2):
          for b in range(2):
            j = 2 * jj + b
            gbuf, sbuf = gbufs[b], sbufs[b]
            gsem, ssem = gsems[b], ssems[b]
            pltpu.make_async_copy(gdummy, gbuf, gsem).wait()
            pltpu.make_async_copy(gdummy, sbuf, ssem).wait()
            scale_chunk(wgt_v, gbuf, sbuf, j)
            pltpu.async_copy(sbuf, acc.at[dst_v.at[j]], ssem, add=True)
            issue_gather(src_v.at[j + 2], b)

          return carry2

        lax.fori_loop(1, p_ch // 2 - 1, chunk_pair, 0)
        for b in range(2):
          j = p_ch - 2 + b
          pltpu.make_async_copy(gdummy, gbufs[b], gsems[b]).wait()
          pltpu.make_async_copy(gdummy, sbufs[b], ssems[b]).wait()
          scale_chunk(wgt_v, gbufs[b], sbufs[b], j)
          pltpu.async_copy(sbufs[b], acc.at[dst_v.at[j]], ssems[b], add=True)
        if sub == 0:
          stage_wait(1)
          issue_gather(srcb1.at[0], 0)
          issue_gather(srcb1.at[1], 1)
        else:
          @pl.when(jnp.logical_not(last_pp))
          def _():
            stage_wait(0)
            issue_gather(srcb0.at[0], 0)
            issue_gather(srcb0.at[1], 1)

      return carry

    lax.fori_loop(0, n_pp, double_phase, 0)

    pltpu.make_async_copy(gdummy, sbuf0, ssem0).wait()
    pltpu.make_async_copy(gdummy, sbuf1, ssem1).wait()
    plsc.subcore_barrier()

    pltpu.sync_copy(acc.at[pl.ds(base, rows_per_sub)],
                    out_hbm.at[cid, pl.ds(base, rows_per_sub)])

  n_acc_ = support2.shape[1]
  return sc_kernel(support2, support2.reshape(NC * n_acc_, DH),
                   src2d, dst2d, wgt2d)


def _finalize_tc(partials, b_gcn, n_nodes, blk):
  n_blocks = n_nodes // blk

  def body(p_ref, b_ref, out_ref):
    s = jnp.concatenate([p_ref[0], p_ref[1]], axis=1) + b_ref[0]
    n = jnp.sqrt(jnp.sum(s * s, axis=1, keepdims=True))
    out_ref[...] = s / jnp.maximum(n, 1e-12)

  return pl.pallas_call(
      body,
      grid=(n_blocks,),
      in_specs=[
          pl.BlockSpec((NC, blk, DH), lambda i: (0, i, 0)),
          pl.BlockSpec((1, 1, D), lambda i: (0, 0, 0)),
      ],
      out_specs=pl.BlockSpec((blk, D), lambda i: (i, 0)),
      out_shape=jax.ShapeDtypeStruct((n_nodes, D), jnp.float32),
  )(partials, b_gcn.reshape(1, 1, D))


def kernel(feat_a, feat_p, feat_s, edge_index, edge_weight,
           W_enc_a, b_enc_a, W_enc_p, b_enc_p, W_enc_s, b_enc_s,
           W_gcn, b_gcn):
  n_a, n_p, n_s = feat_a.shape[0], feat_p.shape[0], feat_s.shape[0]
  n_nodes = n_a + n_p + n_s
  n_edges = edge_weight.shape[0]

  feats = jnp.concatenate([feat_a, feat_p, feat_s], axis=0)
  W_enc = jnp.stack([W_enc_a, W_enc_p, W_enc_s], axis=0)
  b_enc = jnp.stack([b_enc_a, b_enc_p, b_enc_s], axis=0).reshape(3, 1, D)

  blk = 2000

  per_tile_edges = -(-n_edges // NS)
  ch_per_tile = -(-(-(-per_tile_edges // K_EDGES)) // 32) * 32
  e_pad = NS * ch_per_tile * K_EDGES
  n_acc = -(-(n_nodes + 1) // (NS * 8)) * (NS * 8)

  src = jnp.pad(edge_index[0], (0, e_pad - n_edges)).reshape(-1, K_EDGES)
  dst = jnp.pad(edge_index[1], (0, e_pad - n_edges),
                constant_values=n_nodes).reshape(-1, K_EDGES)
  wgt = jnp.pad(edge_weight, (0, e_pad - n_edges)).reshape(-1, K_EDGES)

  support2 = _support_tc(feats, W_enc, b_enc, W_gcn,
                         n_a // blk, (n_a + n_p) // blk, blk, n_acc)

  partials = _edge_aggregate_sc(support2, src, dst, wgt, n_acc, ch_per_tile)

  return _finalize_tc(partials, b_gcn, n_nodes, blk)


# speedup vs baseline: 1.4911x; 1.4911x over previous
import functools

import jax
import jax.numpy as jnp
from jax import lax
from jax.experimental import pallas as pl
from jax.experimental.pallas import tpu as pltpu
from jax.experimental.pallas import tpu_sc as plsc

NC = 2
NS = 16
LANES = 16

K_EDGES = 128
D = 128
DH = D // NC


def _support_tc(feats, W_enc_stacked, b_enc_stacked, W_gcn, b1, b2, blk,
                n_pad):
  n_total = feats.shape[0]
  n_blocks = n_total // blk

  def tmap(i):
    return jnp.where(i >= b2, 2, jnp.where(i >= b1, 1, 0))

  def body(x_ref, w_ref, b_ref, wg_ref, out_ref):
    x = x_ref[...]
    enc = jnp.maximum(
        jnp.dot(x, w_ref[0], preferred_element_type=jnp.float32) + b_ref[0],
        0.0)
    s = jnp.dot(enc, wg_ref[...], preferred_element_type=jnp.float32)
    out_ref[0] = s[:, :DH]
    out_ref[1] = s[:, DH:]

  return pl.pallas_call(
      body,
      grid=(n_blocks,),
      in_specs=[
          pl.BlockSpec((blk, D), lambda i: (i, 0)),
          pl.BlockSpec((1, D, D), lambda i: (tmap(i), 0, 0)),
          pl.BlockSpec((1, 1, D), lambda i: (tmap(i), 0, 0)),
          pl.BlockSpec((D, D), lambda i: (0, 0)),
      ],
      out_specs=pl.BlockSpec((NC, blk, DH), lambda i: (0, i, 0)),
      out_shape=jax.ShapeDtypeStruct((NC, n_pad, DH), jnp.float32),
  )(feats, W_enc_stacked, b_enc_stacked, W_gcn)


def _edge_aggregate_sc(support2, src2d, dst2d, wgt2d, n_acc, ch_per_tile):
  rows_per_sub = n_acc // NS
  p_ch = 16
  n_phase = ch_per_tile // p_ch
  hb = DH // LANES

  mesh = plsc.VectorSubcoreMesh(core_axis_name="c", subcore_axis_name="s")

  @functools.partial(
      pl.kernel,
      out_type=jax.ShapeDtypeStruct((NC, n_acc, DH), jnp.float32),
      mesh=mesh,
      compiler_params=pltpu.CompilerParams(use_tc_tiling_on_sc=False),
      scratch_types=[
          pltpu.VMEM((p_ch, K_EDGES), jnp.int32),
          pltpu.VMEM((p_ch, K_EDGES), jnp.int32),
          pltpu.VMEM((p_ch, K_EDGES), jnp.int32),
          pltpu.VMEM((p_ch, K_EDGES), jnp.int32),
          pltpu.VMEM((p_ch, K_EDGES), jnp.float32),
          pltpu.VMEM((p_ch, K_EDGES), jnp.float32),
          pltpu.VMEM((K_EDGES, DH), jnp.float32),
          pltpu.VMEM((K_EDGES, DH), jnp.float32),
          pltpu.VMEM((K_EDGES, DH), jnp.float32),
          pltpu.VMEM((K_EDGES, DH), jnp.float32),
          pltpu.VMEM_SHARED((n_acc, DH), jnp.float32),
          pltpu.VMEM_SHARED((n_acc, DH), jnp.float32),
          pltpu.SemaphoreType.DMA,
          pltpu.SemaphoreType.DMA,
          pltpu.SemaphoreType.DMA,
          pltpu.SemaphoreType.DMA,
          pltpu.SemaphoreType.DMA,
          pltpu.SemaphoreType.DMA,
      ],
  )
  def sc_kernel(sup_hbm, src_hbm, dst_hbm, wgt_hbm, out_hbm,
                srcb0, srcb1, dstb0, dstb1, wgtb0, wgtb1,
                gbuf0, gbuf1, sbuf0, sbuf1,
                sup_sp, acc, gsem0, gsem1, ssem0, ssem1, slsem0, slsem1):
    cid = lax.axis_index("c")
    sid = lax.axis_index("s")
    gbufs, sbufs = (gbuf0, gbuf1), (sbuf0, sbuf1)
    gsems, ssems = (gsem0, gsem1), (ssem0, ssem1)
    srcbs, dstbs, wgtbs = (srcb0, srcb1), (dstb0, dstb1), (wgtb0, wgtb1)
    slsems = (slsem0, slsem1)
    gdummy = sup_hbm.at[0, pl.ds(0, K_EDGES)]
    sdummy_i = src_hbm.at[pl.ds(0, p_ch)]
    sdummy_f = wgt_hbm.at[pl.ds(0, p_ch)]
    base = sid * rows_per_sub

    pltpu.sync_copy(sup_hbm.at[cid, pl.ds(base, rows_per_sub)],
                    sup_sp.at[pl.ds(base, rows_per_sub)])

    def zfill(i, carry):
      for r in range(hb):
        gbuf0[i, pl.ds(r * LANES, LANES)] = jnp.zeros((LANES,), jnp.float32)
      return carry

    lax.fori_loop(0, K_EDGES, zfill, 0)
    nfull = rows_per_sub // K_EDGES
    rem = rows_per_sub - nfull * K_EDGES
    for kb in range(nfull):
      pltpu.sync_copy(gbuf0.at[pl.ds(0, K_EDGES)],
                      acc.at[pl.ds(base + kb * K_EDGES, K_EDGES)])
    if rem:
      pltpu.sync_copy(gbuf0.at[pl.ds(0, rem)],
                      acc.at[pl.ds(base + nfull * K_EDGES, rem)])
    plsc.subcore_barrier()

    def stage(pi, q):
      eb = sid * ch_per_tile + pi * p_ch
      pltpu.async_copy(src_hbm.at[pl.ds(eb, p_ch)], srcbs[q], slsems[q])
      pltpu.async_copy(dst_hbm.at[pl.ds(eb, p_ch)], dstbs[q], slsems[q])
      pltpu.async_copy(wgt_hbm.at[pl.ds(eb, p_ch)], wgtbs[q], slsems[q])

    def stage_wait(q):
      pltpu.make_async_copy(sdummy_i, srcbs[q], slsems[q]).wait()
      pltpu.make_async_copy(sdummy_i, dstbs[q], slsems[q]).wait()
      pltpu.make_async_copy(sdummy_f, wgtbs[q], slsems[q]).wait()

    def issue_gather(src_row_ref, b):
      pltpu.async_copy(sup_sp.at[src_row_ref], gbufs[b], gsems[b])

    def scale_chunk(wgt_v, gbuf, sbuf, j):
      def gbody(g, c2):
        w16 = wgt_v[j, pl.ds(g * LANES, LANES)]
        for i in range(LANES):
          e = g * LANES + i
          w = w16[i]
          for r in range(hb):
            sl = pl.ds(r * LANES, LANES)
            sbuf[e, sl] = gbuf[e, sl] * w
        return c2

      lax.fori_loop(0, K_EDGES // LANES, gbody, 0)

    stage(0, 0)
    stage_wait(0)
    stage(1, 1)
    issue_gather(srcb0.at[0], 0)
    issue_gather(srcb0.at[1], 1)

    n_pp = n_phase // 2

    def double_phase(pp, carry):
      for sub in range(2):
        q = sub
        p_dyn = 2 * pp + sub
        src_v, dst_v, wgt_v = srcbs[q], dstbs[q], wgtbs[q]
        last_pp = pp >= n_pp - 1

        for b in range(2):
          pltpu.make_async_copy(gdummy, gbufs[b], gsems[b]).wait()
          if sub == 0:
            @pl.when(pp > 0)
            def _():
              pltpu.make_async_copy(gdummy, sbufs[b], ssems[b]).wait()
          else:
            pltpu.make_async_copy(gdummy, sbufs[b], ssems[b]).wait()
          scale_chunk(wgt_v, gbufs[b], sbufs[b], b)
          pltpu.async_copy(sbufs[b], acc.at[dst_v.at[b]], ssems[b], add=True)
          issue_gather(src_v.at[b + 2], b)
        if sub == 0:
          @pl.when(pp > 0)
          def _():
            stage(p_dyn + 1, 1)
        else:
          @pl.when(jnp.logical_not(last_pp))
          def _():
            stage(p_dyn + 1, 0)

        def chunk_pair(jj, carry2):
          for b in range(2):
            j = 2 * jj + b
            gbuf, sbuf = gbufs[b], sbufs[b]
            gsem, ssem = gsems[b], ssems[b]
            pltpu.make_async_copy(gdummy, gbuf, gsem).wait()
            pltpu.make_async_copy(gdummy, sbuf, ssem).wait()
            scale_chunk(wgt_v, gbuf, sbuf, j)
            pltpu.async_copy(sbuf, acc.at[dst_v.at[j]], ssem, add=True)
            issue_gather(src_v.at[j + 2], b)

          return carry2

        lax.fori_loop(1, p_ch // 2 - 1, chunk_pair, 0)
        for b in range(2):
          j = p_ch - 2 + b
          pltpu.make_async_copy(gdummy, gbufs[b], gsems[b]).wait()
          pltpu.make_async_copy(gdummy, sbufs[b], ssems[b]).wait()
          scale_chunk(wgt_v, gbufs[b], sbufs[b], j)
          pltpu.async_copy(sbufs[b], acc.at[dst_v.at[j]], ssems[b], add=True)
        if sub == 0:
          stage_wait(1)
          issue_gather(srcb1.at[0], 0)
          issue_gather(srcb1.at[1], 1)
        else:
          @pl.when(jnp.logical_not(last_pp))
          def _():
            stage_wait(0)
            issue_gather(srcb0.at[0], 0)
            issue_gather(srcb0.at[1], 1)

      return carry

    lax.fori_loop(0, n_pp, double_phase, 0)

    pltpu.make_async_copy(gdummy, sbuf0, ssem0).wait()
    pltpu.make_async_copy(gdummy, sbuf1, ssem1).wait()
    plsc.subcore_barrier()

    pltpu.sync_copy(acc.at[pl.ds(base, rows_per_sub)],
                    out_hbm.at[cid, pl.ds(base, rows_per_sub)])

  return sc_kernel(support2, src2d, dst2d, wgt2d)


def _finalize_tc(partials, b_gcn, n_nodes, blk):
  n_blocks = n_nodes // blk

  def body(p_ref, b_ref, out_ref):
    s = jnp.concatenate([p_ref[0], p_ref[1]], axis=1) + b_ref[0]
    n = jnp.sqrt(jnp.sum(s * s, axis=1, keepdims=True))
    out_ref[...] = s / jnp.maximum(n, 1e-12)

  return pl.pallas_call(
      body,
      grid=(n_blocks,),
      in_specs=[
          pl.BlockSpec((NC, blk, DH), lambda i: (0, i, 0)),
          pl.BlockSpec((1, 1, D), lambda i: (0, 0, 0)),
      ],
      out_specs=pl.BlockSpec((blk, D), lambda i: (i, 0)),
      out_shape=jax.ShapeDtypeStruct((n_nodes, D), jnp.float32),
  )(partials, b_gcn.reshape(1, 1, D))


def kernel(feat_a, feat_p, feat_s, edge_index, edge_weight,
           W_enc_a, b_enc_a, W_enc_p, b_enc_p, W_enc_s, b_enc_s,
           W_gcn, b_gcn):
  n_a, n_p, n_s = feat_a.shape[0], feat_p.shape[0], feat_s.shape[0]
  n_nodes = n_a + n_p + n_s
  n_edges = edge_weight.shape[0]

  feats = jnp.concatenate([feat_a, feat_p, feat_s], axis=0)
  W_enc = jnp.stack([W_enc_a, W_enc_p, W_enc_s], axis=0)
  b_enc = jnp.stack([b_enc_a, b_enc_p, b_enc_s], axis=0).reshape(3, 1, D)

  blk = 2000

  per_tile_edges = -(-n_edges // NS)
  ch_per_tile = -(-(-(-per_tile_edges // K_EDGES)) // 32) * 32
  e_pad = NS * ch_per_tile * K_EDGES
  n_acc = -(-(n_nodes + 1) // (NS * 8)) * (NS * 8)

  src = jnp.pad(edge_index[0], (0, e_pad - n_edges)).reshape(-1, K_EDGES)
  dst = jnp.pad(edge_index[1], (0, e_pad - n_edges),
                constant_values=n_nodes).reshape(-1, K_EDGES)
  wgt = jnp.pad(edge_weight, (0, e_pad - n_edges)).reshape(-1, K_EDGES)

  support2 = _support_tc(feats, W_enc, b_enc, W_gcn,
                         n_a // blk, (n_a + n_p) // blk, blk, n_acc)

  partials = _edge_aggregate_sc(support2, src, dst, wgt, n_acc, ch_per_tile)

  return _finalize_tc(partials, b_gcn, n_nodes, blk)
